# baseline (speedup 1.0000x reference)
import jax
import jax.numpy as jnp
from jax.experimental import pallas as pl
from jax.experimental.pallas import tpu as pltpu

B = 2048
OBS_DIM = 768
ACT_DIM = 32
OPTION_DIM = 4
MOE_H = 1536
E = 8
K = 2

TB = 256


def _fused_kernel(obs_ref, wg_ref, w1_ref, b1_ref, w2_ref, b2_ref,
                  lw_ref, lb_ref, gates_ref, y_ref, p_ref, lp_ref):
    x = obs_ref[...]

    logits = jnp.dot(x, wg_ref[...], preferred_element_type=jnp.float32)
    idx = jax.lax.broadcasted_iota(jnp.int32, (TB, E), 1)
    m1 = jnp.max(logits, axis=1, keepdims=True)
    a1 = jnp.min(jnp.where(logits == m1, idx, E), axis=1, keepdims=True)
    oh1 = idx == a1
    l2 = jnp.where(oh1, -jnp.inf, logits)
    m2 = jnp.max(l2, axis=1, keepdims=True)
    a2 = jnp.min(jnp.where(l2 == m2, idx, E), axis=1, keepdims=True)
    oh2 = idx == a2
    e2 = jnp.exp(m2 - m1)
    denom = 1.0 + e2
    gates = jnp.where(oh1, 1.0 / denom, jnp.where(oh2, e2 / denom, 0.0))
    gates_ref[...] = gates

    y = jnp.zeros((TB, ACT_DIM), jnp.float32)
    for e in range(E):
        y = y + gates[:, e][:, None] * b2_ref[e]
    for e in range(E):
        h = jnp.dot(x, w1_ref[e], preferred_element_type=jnp.float32)
        h = jnp.maximum(h + b1_ref[e], 0.0)
        hg = h * gates[:, e][:, None]
        y = y + jnp.dot(hg.astype(jnp.bfloat16), w2_ref[e],
                        preferred_element_type=jnp.float32)
    y_ref[...] = y

    for o in range(OPTION_DIM):
        mu = jnp.dot(x, lw_ref[o], preferred_element_type=jnp.float32)
        mu = mu + lb_ref[o][None, :]
        mx = jnp.max(mu, axis=-1, keepdims=True)
        ex = jnp.exp(mu - mx)
        p = ex / jnp.sum(ex, axis=-1, keepdims=True)
        z = jnp.where(p == 0.0, 1e-8, 0.0)
        p_ref[o] = p
        lp_ref[o] = jnp.log(p + z)


@jax.jit
def kernel(obs, w_gate, W1, b1, W2, b2, last_w, last_b):
    nbt = B // TB
    obs16 = obs.astype(jnp.bfloat16)
    wg16 = w_gate[0].astype(jnp.bfloat16)
    W1_16 = W1.astype(jnp.bfloat16)
    W2_16 = W2.astype(jnp.bfloat16)
    lw16 = last_w.astype(jnp.bfloat16)

    gates, y, action_probs, log_probs = pl.pallas_call(
        _fused_kernel,
        grid=(nbt,),
        in_specs=[
            pl.BlockSpec((TB, OBS_DIM), lambda i: (i, 0)),
            pl.BlockSpec((OBS_DIM, E), lambda i: (0, 0)),
            pl.BlockSpec((E, OBS_DIM, MOE_H), lambda i: (0, 0, 0)),
            pl.BlockSpec((E, 1, MOE_H), lambda i: (0, 0, 0)),
            pl.BlockSpec((E, MOE_H, ACT_DIM), lambda i: (0, 0, 0)),
            pl.BlockSpec((E, ACT_DIM), lambda i: (0, 0)),
            pl.BlockSpec((OPTION_DIM, OBS_DIM, ACT_DIM), lambda i: (0, 0, 0)),
            pl.BlockSpec((OPTION_DIM, ACT_DIM), lambda i: (0, 0)),
        ],
        out_specs=[
            pl.BlockSpec((TB, E), lambda i: (i, 0)),
            pl.BlockSpec((TB, ACT_DIM), lambda i: (i, 0)),
            pl.BlockSpec((OPTION_DIM, TB, ACT_DIM), lambda i: (0, i, 0)),
            pl.BlockSpec((OPTION_DIM, TB, ACT_DIM), lambda i: (0, i, 0)),
        ],
        out_shape=[
            jax.ShapeDtypeStruct((B, E), jnp.float32),
            jax.ShapeDtypeStruct((B, ACT_DIM), jnp.float32),
            jax.ShapeDtypeStruct((OPTION_DIM, B, ACT_DIM), jnp.float32),
            jax.ShapeDtypeStruct((OPTION_DIM, B, ACT_DIM), jnp.float32),
        ],
    )(obs16, wg16, W1_16, b1[:, None, :], W2_16, b2, lw16, last_b)

    return (action_probs, log_probs, y)


# speedup vs baseline: 1.1485x; 1.1485x over previous
import jax
import jax.numpy as jnp
from jax.experimental import pallas as pl
from jax.experimental.pallas import tpu as pltpu

B = 2048
OBS_DIM = 768
ACT_DIM = 32
OPTION_DIM = 4
MOE_H = 1536
E = 8
K = 2

TB = 256


def _fused_kernel(obs_ref, wg_ref, w1_ref, b1_ref, w2_ref, b2_ref,
                  lw_ref, lb_ref, gates_ref, y_ref, p_ref, lp_ref):
    x = obs_ref[...].astype(jnp.bfloat16)

    logits = jnp.dot(x, wg_ref[...].astype(jnp.bfloat16),
                     preferred_element_type=jnp.float32)
    idx = jax.lax.broadcasted_iota(jnp.int32, (TB, E), 1)
    m1 = jnp.max(logits, axis=1, keepdims=True)
    a1 = jnp.min(jnp.where(logits == m1, idx, E), axis=1, keepdims=True)
    oh1 = idx == a1
    l2 = jnp.where(oh1, -jnp.inf, logits)
    m2 = jnp.max(l2, axis=1, keepdims=True)
    a2 = jnp.min(jnp.where(l2 == m2, idx, E), axis=1, keepdims=True)
    oh2 = idx == a2
    e2 = jnp.exp(m2 - m1)
    denom = 1.0 + e2
    gates = jnp.where(oh1, 1.0 / denom, jnp.where(oh2, e2 / denom, 0.0))
    gates_ref[...] = gates

    y = jnp.zeros((TB, ACT_DIM), jnp.float32)
    for e in range(E):
        y = y + gates[:, e][:, None] * b2_ref[e]
    for e in range(E):
        h = jnp.dot(x, w1_ref[e].astype(jnp.bfloat16),
                    preferred_element_type=jnp.float32)
        h = jnp.maximum(h + b1_ref[e], 0.0)
        hg = h * gates[:, e][:, None]
        y = y + jnp.dot(hg.astype(jnp.bfloat16), w2_ref[e].astype(jnp.bfloat16),
                        preferred_element_type=jnp.float32)
    y_ref[...] = y

    for o in range(OPTION_DIM):
        mu = jnp.dot(x, lw_ref[o].astype(jnp.bfloat16),
                     preferred_element_type=jnp.float32)
        mu = mu + lb_ref[o][None, :]
        mx = jnp.max(mu, axis=-1, keepdims=True)
        ex = jnp.exp(mu - mx)
        p = ex / jnp.sum(ex, axis=-1, keepdims=True)
        z = jnp.where(p == 0.0, 1e-8, 0.0)
        p_ref[o] = p
        lp_ref[o] = jnp.log(p + z)


@jax.jit
def kernel(obs, w_gate, W1, b1, W2, b2, last_w, last_b):
    nbt = B // TB

    gates, y, action_probs, log_probs = pl.pallas_call(
        _fused_kernel,
        grid=(nbt,),
        in_specs=[
            pl.BlockSpec((TB, OBS_DIM), lambda i: (i, 0)),
            pl.BlockSpec((OBS_DIM, E), lambda i: (0, 0)),
            pl.BlockSpec((E, OBS_DIM, MOE_H), lambda i: (0, 0, 0)),
            pl.BlockSpec((E, 1, MOE_H), lambda i: (0, 0, 0)),
            pl.BlockSpec((E, MOE_H, ACT_DIM), lambda i: (0, 0, 0)),
            pl.BlockSpec((E, ACT_DIM), lambda i: (0, 0)),
            pl.BlockSpec((OPTION_DIM, OBS_DIM, ACT_DIM), lambda i: (0, 0, 0)),
            pl.BlockSpec((OPTION_DIM, ACT_DIM), lambda i: (0, 0)),
        ],
        out_specs=[
            pl.BlockSpec((TB, E), lambda i: (i, 0)),
            pl.BlockSpec((TB, ACT_DIM), lambda i: (i, 0)),
            pl.BlockSpec((OPTION_DIM, TB, ACT_DIM), lambda i: (0, i, 0)),
            pl.BlockSpec((OPTION_DIM, TB, ACT_DIM), lambda i: (0, i, 0)),
        ],
        out_shape=[
            jax.ShapeDtypeStruct((B, E), jnp.float32),
            jax.ShapeDtypeStruct((B, ACT_DIM), jnp.float32),
            jax.ShapeDtypeStruct((OPTION_DIM, B, ACT_DIM), jnp.float32),
            jax.ShapeDtypeStruct((OPTION_DIM, B, ACT_DIM), jnp.float32),
        ],
    )(obs, w_gate[0], W1, b1[:, None, :], W2, b2, last_w, last_b)

    return (action_probs, log_probs, y)
